# baseline (speedup 1.0000x reference)
import jax
import jax.numpy as jnp
from jax.experimental import pallas as pl
from jax.experimental.pallas import tpu as pltpu

K1 = 33
N = 4096
D = 256
BR = 256
NB = N // BR
_INF_BITS = 0x7F800000


def _dtm_kernel(xf_ref, xb_ref, loss_ref, sprev_ref):
    i = pl.program_id(0)
    m = pl.program_id(1)
    xb = xb_ref[0]
    xf = xf_ref[0]
    a2 = jnp.sum(xb * xb, axis=1, keepdims=True)
    b2 = jnp.sum(xf * xf, axis=1)[None, :]
    g = jax.lax.dot_general(xb, xf, (((1,), (1,)), ((), ())),
                            preferred_element_type=jnp.float32)
    d2 = jnp.maximum(a2 + b2 - 2.0 * g, 0.0)
    bits = jax.lax.bitcast_convert_type(d2, jnp.int32)

    gbits = jnp.min(bits.reshape(BR, N // 128, 128), axis=1)

    def gsel(kth):
        def gbody(_, carry):
            lo, hi = carry
            mid = lo + (hi - lo) // 2
            cnt = jnp.sum((gbits <= mid).astype(jnp.int32), axis=1,
                          keepdims=True)
            ge = cnt >= kth
            return jnp.where(ge, lo, mid + 1), jnp.where(ge, mid, hi)
        z = jnp.zeros((BR, 1), jnp.int32)
        h = jnp.full((BR, 1), _INF_BITS, jnp.int32)
        return jax.lax.fori_loop(0, 31, gbody, (z, h))[1]

    lo0 = gsel(2)
    hi0 = gsel(K1)

    def cond(carry):
        lo, hi = carry
        return jnp.any(lo < hi)

    def step(_, carry):
        lo, hi = carry
        mid = lo + (hi - lo) // 2
        cnt = jnp.sum((bits <= mid).astype(jnp.int32), axis=1, keepdims=True)
        ge = cnt >= K1
        return jnp.where(ge, lo, mid + 1), jnp.where(ge, mid, hi)

    _, tbits = jax.lax.fori_loop(0, 31, lambda j, c: step(j, c), (lo0, hi0))
    t = jax.lax.bitcast_convert_type(tbits, jnp.float32)

    dist = jnp.sqrt(d2)
    lt = bits < tbits
    cnt_lt = jnp.sum(lt.astype(jnp.float32), axis=1, keepdims=True)
    sum_lt = jnp.sum(jnp.where(lt, dist, 0.0), axis=1, keepdims=True)
    s = sum_lt + (K1 - cnt_lt) * jnp.sqrt(t)

    @pl.when(jnp.logical_and(i == 0, m == 0))
    def _():
        loss_ref[:, :] = jnp.zeros((1, 1), jnp.float32)

    @pl.when(m == 0)
    def _():
        sprev_ref[:, :] = s

    @pl.when(m == 1)
    def _():
        diff = s - sprev_ref[:, :]
        loss_ref[:, :] += jnp.sum(diff * diff).reshape(1, 1)

    @pl.when(jnp.logical_and(i == NB - 1, m == 1))
    def _():
        loss_ref[:, :] = loss_ref[:, :] / N


def kernel(x_1, x_2):
    xs = jnp.stack([x_1, x_2])
    out = pl.pallas_call(
        _dtm_kernel,
        grid=(NB, 2),
        in_specs=[
            pl.BlockSpec((1, N, D), lambda i, m: (m, 0, 0)),
            pl.BlockSpec((1, BR, D), lambda i, m: (m, i, 0)),
        ],
        out_specs=pl.BlockSpec((1, 1), lambda i, m: (0, 0)),
        out_shape=jax.ShapeDtypeStruct((1, 1), jnp.float32),
        scratch_shapes=[pltpu.VMEM((BR, 1), jnp.float32)],
    )(xs, xs)
    return out[0, 0]


# speedup vs baseline: 1.4419x; 1.4419x over previous
import jax
import jax.numpy as jnp
from jax.experimental import pallas as pl
from jax.experimental.pallas import tpu as pltpu

K1 = 33
N = 4096
D = 256
BR = 256
NB = N // BR
_INF_BITS = 0x7F800000


def _dtm_kernel(xf_ref, xb_ref, loss_ref, sprev_ref):
    i = pl.program_id(0)
    m = pl.program_id(1)
    xb = xb_ref[0]
    xf = xf_ref[0]
    a2 = jnp.sum(xb * xb, axis=1, keepdims=True)
    b2 = jnp.sum(xf * xf, axis=1)[None, :]
    g = jax.lax.dot_general(xb, xf, (((1,), (1,)), ((), ())),
                            preferred_element_type=jnp.float32)
    d2 = jnp.maximum(a2 + b2 - 2.0 * g, 0.0)
    bits = jax.lax.bitcast_convert_type(d2, jnp.int32)

    gbits = bits[:, 0:128]
    for j in range(1, N // 128):
        gbits = jnp.minimum(gbits, bits[:, j * 128:(j + 1) * 128])

    def gsel(kth):
        def gbody(_, carry):
            lo, hi = carry
            mid = lo + (hi - lo) // 2
            cnt = jnp.sum((gbits <= mid).astype(jnp.int32), axis=1,
                          keepdims=True)
            ge = cnt >= kth
            return jnp.where(ge, lo, mid + 1), jnp.where(ge, mid, hi)
        z = jnp.zeros((BR, 1), jnp.int32)
        h = jnp.full((BR, 1), _INF_BITS, jnp.int32)
        return jax.lax.fori_loop(0, 31, gbody, (z, h))[1]

    lo0 = gsel(2)
    hi0 = gsel(K1)

    def cond(carry):
        lo, hi = carry
        return jnp.any(lo < hi)

    def step(_, carry):
        lo, hi = carry
        mid = lo + (hi - lo) // 2
        cnt = jnp.sum((bits <= mid).astype(jnp.int32), axis=1, keepdims=True)
        ge = cnt >= K1
        return jnp.where(ge, lo, mid + 1), jnp.where(ge, mid, hi)

    def body(carry):
        return jax.lax.fori_loop(0, 8, step, carry)

    _, tbits = jax.lax.while_loop(cond, body, (lo0, hi0))
    t = jax.lax.bitcast_convert_type(tbits, jnp.float32)

    dist = jnp.sqrt(d2)
    lt = bits < tbits
    cnt_lt = jnp.sum(lt.astype(jnp.float32), axis=1, keepdims=True)
    sum_lt = jnp.sum(jnp.where(lt, dist, 0.0), axis=1, keepdims=True)
    s = sum_lt + (K1 - cnt_lt) * jnp.sqrt(t)

    @pl.when(jnp.logical_and(i == 0, m == 0))
    def _():
        loss_ref[:, :] = jnp.zeros((1, 1), jnp.float32)

    @pl.when(m == 0)
    def _():
        sprev_ref[:, :] = s

    @pl.when(m == 1)
    def _():
        diff = s - sprev_ref[:, :]
        loss_ref[:, :] += jnp.sum(diff * diff).reshape(1, 1)

    @pl.when(jnp.logical_and(i == NB - 1, m == 1))
    def _():
        loss_ref[:, :] = loss_ref[:, :] / N


def kernel(x_1, x_2):
    xs = jnp.stack([x_1, x_2])
    out = pl.pallas_call(
        _dtm_kernel,
        grid=(NB, 2),
        in_specs=[
            pl.BlockSpec((1, N, D), lambda i, m: (m, 0, 0)),
            pl.BlockSpec((1, BR, D), lambda i, m: (m, i, 0)),
        ],
        out_specs=pl.BlockSpec((1, 1), lambda i, m: (0, 0)),
        out_shape=jax.ShapeDtypeStruct((1, 1), jnp.float32),
        scratch_shapes=[pltpu.VMEM((BR, 1), jnp.float32)],
    )(xs, xs)
    return out[0, 0]


# speedup vs baseline: 2.4415x; 1.6932x over previous
import jax
import jax.numpy as jnp
from jax.experimental import pallas as pl
from jax.experimental.pallas import tpu as pltpu

K1 = 33
N = 4096
D = 256
BR = 256
NB = N // BR
_INF_BITS = 0x7F800000


def _dtm_kernel(xf_ref, xb_ref, loss_ref, sprev_ref):
    i = pl.program_id(0)
    m = pl.program_id(1)
    xb = xb_ref[0]
    xf = xf_ref[0]
    a2 = jnp.sum(xb * xb, axis=1, keepdims=True)
    b2 = jnp.sum(xf * xf, axis=1)[None, :]
    g = jax.lax.dot_general(xb, xf, (((1,), (1,)), ((), ())),
                            preferred_element_type=jnp.float32)
    d2 = jnp.maximum(a2 + b2 - 2.0 * g, 0.0)
    bits = jax.lax.bitcast_convert_type(d2, jnp.int32)

    gbits = bits[:, 0:128]
    for j in range(1, N // 128):
        gbits = jnp.minimum(gbits, bits[:, j * 128:(j + 1) * 128])

    lane = jax.lax.broadcasted_iota(jnp.int32, (BR, 128), 1)
    row = jax.lax.broadcasted_iota(jnp.int32, (BR, 128), 0)
    diag_lane = (i * BR + row) % 128
    gm_nd = jnp.where(lane == diag_lane, _INF_BITS, gbits)
    lo0 = jnp.min(gm_nd, axis=1, keepdims=True)
    hi0 = jnp.max(gbits, axis=1, keepdims=True)

    def cond(carry):
        lo, hi = carry
        return jnp.any(lo < hi)

    def step(_, carry):
        lo, hi = carry
        mid = lo + (hi - lo) // 2
        cnt = jnp.sum((bits <= mid).astype(jnp.int32), axis=1, keepdims=True)
        ge = cnt >= K1
        return jnp.where(ge, lo, mid + 1), jnp.where(ge, mid, hi)

    def body(carry):
        return jax.lax.fori_loop(0, 8, step, carry)

    _, tbits = jax.lax.while_loop(cond, body, (lo0, hi0))
    t = jax.lax.bitcast_convert_type(tbits, jnp.float32)

    dist = jnp.sqrt(d2)
    lt = bits < tbits
    cnt_lt = jnp.sum(lt.astype(jnp.float32), axis=1, keepdims=True)
    sum_lt = jnp.sum(jnp.where(lt, dist, 0.0), axis=1, keepdims=True)
    s = sum_lt + (K1 - cnt_lt) * jnp.sqrt(t)

    @pl.when(jnp.logical_and(i == 0, m == 0))
    def _():
        loss_ref[:, :] = jnp.zeros((1, 1), jnp.float32)

    @pl.when(m == 0)
    def _():
        sprev_ref[:, :] = s

    @pl.when(m == 1)
    def _():
        diff = s - sprev_ref[:, :]
        loss_ref[:, :] += jnp.sum(diff * diff).reshape(1, 1)

    @pl.when(jnp.logical_and(i == NB - 1, m == 1))
    def _():
        loss_ref[:, :] = loss_ref[:, :] / N


def kernel(x_1, x_2):
    xs = jnp.stack([x_1, x_2])
    out = pl.pallas_call(
        _dtm_kernel,
        grid=(NB, 2),
        in_specs=[
            pl.BlockSpec((1, N, D), lambda i, m: (m, 0, 0)),
            pl.BlockSpec((1, BR, D), lambda i, m: (m, i, 0)),
        ],
        out_specs=pl.BlockSpec((1, 1), lambda i, m: (0, 0)),
        out_shape=jax.ShapeDtypeStruct((1, 1), jnp.float32),
        scratch_shapes=[pltpu.VMEM((BR, 1), jnp.float32)],
    )(xs, xs)
    return out[0, 0]
